# baseline (device time: 56149 ns/iter reference)
import functools

import jax
import jax.numpy as jnp
from jax import lax
from jax.experimental import pallas as pl
from jax.experimental.pallas import tpu as pltpu

N_DEV = 8

GROUPS = (
    ((1, 2, 4), 0, 640),
    ((2, 4, 1), 640, 640),
    ((4, 1, 2), 1280, 768),
)


def kernel(x, w_mat, scale_x, scale_w):
    m_global, k_per = x.shape
    k_per2, n = w_mat.shape
    assert k_per == k_per2
    m_per = m_global // N_DEV

    def body(x_ref, w_ref, sx_ref, sw_ref, out_ref,
             a0_ref, a1_ref, a2_ref, b0_ref, b1_ref, b2_ref,
             c0_ref, c1_ref, c2_ref, d0_ref, d1_ref, d2_ref,
             acc_ref, keep_ref, send_sems, recv_sems):
        p0_send = (a0_ref, a1_ref, a2_ref)
        p0_recv = (b0_ref, b1_ref, b2_ref)
        p12_send = (c0_ref, c1_ref, c2_ref)
        p12_recv = (d0_ref, d1_ref, d2_ref)

        def slot_refs(gi, slot):
            if slot < 4:
                return p0_send[gi].at[slot], p0_recv[gi].at[slot]
            return p12_send[gi].at[slot - 4], p12_recv[gi].at[slot - 4]

        my_pos = lax.axis_index("i")

        def gray(t):
            return (t & 4) | ((t & 3) ^ ((t & 3) >> 1))

        my_v = gray(my_pos)

        def partner(e):
            return gray(my_v ^ e)

        barrier_sem = pltpu.get_barrier_semaphore()
        for e in (1, 2, 4):
            pl.semaphore_signal(
                barrier_sem, inc=1,
                device_id=(partner(e),), device_id_type=pl.DeviceIdType.MESH,
            )

        def local_chunk(rho, off, w):
            c = gray(my_v ^ rho)
            return jax.lax.dot_general(
                x_ref[pl.ds(c * m_per, m_per), :],
                w_ref[:, pl.ds(off, w)],
                dimension_numbers=(((1,), (0,)), ((), ())),
                preferred_element_type=jnp.int32,
            )

        started = []

        def start(gi, slot, dst_e):
            src, dst = slot_refs(gi, slot)
            rdma = pltpu.make_async_remote_copy(
                src_ref=src,
                dst_ref=dst,
                send_sem=send_sems.at[gi, slot],
                recv_sem=recv_sems.at[gi, slot],
                device_id=(partner(dst_e),),
                device_id_type=pl.DeviceIdType.MESH,
            )
            rdma.start()
            started.append(rdma)

        def wait_recv(gi, slot):
            src, dst = slot_refs(gi, slot)
            pltpu.make_async_remote_copy(
                src_ref=src,
                dst_ref=dst,
                send_sem=send_sems.at[gi, slot],
                recv_sem=recv_sems.at[gi, slot],
                device_id=(partner(GROUPS[gi][0][0]),),
                device_id_type=pl.DeviceIdType.MESH,
            ).wait_recv()

        first = True
        for k in range(4):
            for gi, ((d0, d1, d2), off, w) in enumerate(GROUPS):
                rho_send = (d0 ^ d1 ^ d2, d0 ^ d1, d0 ^ d2, d0)
                p_int = local_chunk(rho_send[k], off, w)
                p0_send[gi][k, :, :] = jnp.clip(
                    (p_int + 2048) >> 12, -127, 127).astype(jnp.int8)
                if first:
                    pl.semaphore_wait(barrier_sem, 3)
                    first = False
                start(gi, k, d0)

        lc = []
        for (d0, d1, d2), off, w in GROUPS:
            lc.append([
                local_chunk(r, off, w).astype(jnp.float32)
                for r in (d1 ^ d2, d1, d2, 0)
            ])

        scale = sx_ref[0] * sw_ref[0]

        def silu(acc_f32):
            y = acc_f32 * scale
            return y * jax.nn.sigmoid(jnp.clip(y, -60.0, 60.0))

        for k in range(4):
            for gi, ((d0, d1, d2), off, w) in enumerate(GROUPS):
                wait_recv(gi, k)
                sum2 = lc[gi][k] + (
                    p0_recv[gi][k, :, :].astype(jnp.float32) * 4096.0)
                if k < 2:
                    p12_send[gi][0 + k, :, :] = sum2.astype(jnp.bfloat16)
                    start(gi, 4 + k, d1)
                elif k == 2:
                    keep_ref[:, pl.ds(off, w)] = sum2
                else:
                    acc_ref[:, pl.ds(off, w)] = sum2

        for gi, ((d0, d1, d2), off, w) in enumerate(GROUPS):
            wait_recv(gi, 4)
            sum4 = (keep_ref[:, pl.ds(off, w)]
                    + p12_recv[gi][0, :, :].astype(jnp.float32))
            p12_send[gi][2, :, :] = sum4.astype(jnp.bfloat16)
            start(gi, 6, d2)
        for gi, ((d0, d1, d2), off, w) in enumerate(GROUPS):
            wait_recv(gi, 5)
            acc_ref[:, pl.ds(off, w)] = (
                acc_ref[:, pl.ds(off, w)]
                + p12_recv[gi][1, :, :].astype(jnp.float32))

        for gi, ((d0, d1, d2), off, w) in enumerate(GROUPS):
            wait_recv(gi, 6)
            final = (acc_ref[:, pl.ds(off, w)]
                     + p12_recv[gi][2, :, :].astype(jnp.float32))
            out_ref[:, pl.ds(off, w)] = silu(final)

        for rdma in started:
            rdma.wait_send()

        @functools.partial(
            pl.run_scoped, second_barrier=pltpu.SemaphoreType.REGULAR)
        def _(second_barrier):
            for e in (1, 2, 4):
                pl.semaphore_signal(
                    second_barrier, inc=1,
                    device_id=(partner(e),),
                    device_id_type=pl.DeviceIdType.MESH,
                )
            pl.semaphore_wait(second_barrier, 3)

    p0_shapes = [
        pltpu.VMEM((4, m_per, w), jnp.int8) for _, _, w in GROUPS
    ]
    p12_shapes = [
        pltpu.VMEM((3, m_per, w), jnp.bfloat16) for _, _, w in GROUPS
    ]
    return pl.pallas_call(
        body,
        out_shape=jax.ShapeDtypeStruct((m_per, n), jnp.float32),
        in_specs=[
            pl.BlockSpec(memory_space=pltpu.VMEM),
            pl.BlockSpec(memory_space=pltpu.VMEM),
            pl.BlockSpec(memory_space=pltpu.SMEM),
            pl.BlockSpec(memory_space=pltpu.SMEM),
        ],
        out_specs=pl.BlockSpec(memory_space=pltpu.VMEM),
        scratch_shapes=p0_shapes + p0_shapes + p12_shapes + p12_shapes + [
            pltpu.VMEM((m_per, n), jnp.float32),
            pltpu.VMEM((m_per, n), jnp.float32),
            pltpu.SemaphoreType.DMA((3, 7)),
            pltpu.SemaphoreType.DMA((3, 7)),
        ],
        compiler_params=pltpu.CompilerParams(
            collective_id=0,
            vmem_limit_bytes=100 * 1024 * 1024,
        ),
    )(x, w_mat, scale_x, scale_w)


# device time: 49124 ns/iter; 1.1430x vs baseline; 1.1430x over previous
import functools

import jax
import jax.numpy as jnp
from jax import lax
from jax.experimental import pallas as pl
from jax.experimental.pallas import tpu as pltpu

N_DEV = 8

GROUPS = (
    ((1, 2, 4), 0, 640),
    ((2, 4, 1), 640, 640),
    ((4, 1, 2), 1280, 768),
)


def kernel(x, w_mat, scale_x, scale_w):
    m_global, k_per = x.shape
    k_per2, n = w_mat.shape
    assert k_per == k_per2
    m_per = m_global // N_DEV

    def body(x_ref, w_ref, sx_ref, sw_ref, out_ref,
             a0_ref, a1_ref, a2_ref, b0_ref, b1_ref, b2_ref,
             c0_ref, c1_ref, c2_ref, d0_ref, d1_ref, d2_ref,
             e0_ref, e1_ref, e2_ref, f0_ref, f1_ref, f2_ref,
             acc_ref, keep_ref, send_sems, recv_sems):
        p0_send = (a0_ref, a1_ref, a2_ref)
        p0_recv = (b0_ref, b1_ref, b2_ref)
        p1_send = (c0_ref, c1_ref, c2_ref)
        p1_recv = (d0_ref, d1_ref, d2_ref)
        p2_send = (e0_ref, e1_ref, e2_ref)
        p2_recv = (f0_ref, f1_ref, f2_ref)

        def slot_refs(gi, slot):
            if slot < 4:
                return p0_send[gi].at[slot], p0_recv[gi].at[slot]
            if slot < 6:
                return p1_send[gi].at[slot - 4], p1_recv[gi].at[slot - 4]
            return p2_send[gi].at[0], p2_recv[gi].at[0]

        my_pos = lax.axis_index("i")

        def gray(t):
            return (t & 4) | ((t & 3) ^ ((t & 3) >> 1))

        my_v = gray(my_pos)

        def partner(e):
            return gray(my_v ^ e)

        barrier_sem = pltpu.get_barrier_semaphore()
        for e in (1, 2, 4):
            pl.semaphore_signal(
                barrier_sem, inc=1,
                device_id=(partner(e),), device_id_type=pl.DeviceIdType.MESH,
            )

        def local_chunk(rho, off, w):
            c = gray(my_v ^ rho)
            return jax.lax.dot_general(
                x_ref[pl.ds(c * m_per, m_per), :],
                w_ref[:, pl.ds(off, w)],
                dimension_numbers=(((1,), (0,)), ((), ())),
                preferred_element_type=jnp.int32,
            )

        started = []

        def start(gi, slot, dst_e):
            src, dst = slot_refs(gi, slot)
            rdma = pltpu.make_async_remote_copy(
                src_ref=src,
                dst_ref=dst,
                send_sem=send_sems.at[gi, slot],
                recv_sem=recv_sems.at[gi, slot],
                device_id=(partner(dst_e),),
                device_id_type=pl.DeviceIdType.MESH,
            )
            rdma.start()
            started.append(rdma)

        def wait_recv(gi, slot):
            src, dst = slot_refs(gi, slot)
            pltpu.make_async_remote_copy(
                src_ref=src,
                dst_ref=dst,
                send_sem=send_sems.at[gi, slot],
                recv_sem=recv_sems.at[gi, slot],
                device_id=(partner(GROUPS[gi][0][0]),),
                device_id_type=pl.DeviceIdType.MESH,
            ).wait_recv()

        first = True
        for k in range(4):
            for gi, ((d0, d1, d2), off, w) in enumerate(GROUPS):
                rho_send = (d0 ^ d1 ^ d2, d0 ^ d1, d0 ^ d2, d0)
                p_int = local_chunk(rho_send[k], off, w)
                p0_send[gi][k, :, :] = jnp.clip(
                    (p_int + 2048) >> 12, -127, 127).astype(jnp.int8)
                if first:
                    pl.semaphore_wait(barrier_sem, 3)
                    first = False
                start(gi, k, d0)

        lc = []
        for (d0, d1, d2), off, w in GROUPS:
            lc.append([
                local_chunk(r, off, w).astype(jnp.float32)
                for r in (d1 ^ d2, d1, d2, 0)
            ])

        scale = sx_ref[0] * sw_ref[0]

        def silu(acc_f32):
            y = acc_f32 * scale
            return y * jax.nn.sigmoid(jnp.clip(y, -60.0, 60.0))

        for k in range(4):
            for gi, ((d0, d1, d2), off, w) in enumerate(GROUPS):
                wait_recv(gi, k)
                sum2 = lc[gi][k] + (
                    p0_recv[gi][k, :, :].astype(jnp.float32) * 4096.0)
                if k < 2:
                    p1_send[gi][k, :, :] = jnp.clip(
                        jnp.round(sum2 * (1.0 / 8192.0)),
                        -127, 127).astype(jnp.int8)
                    start(gi, 4 + k, d1)
                elif k == 2:
                    keep_ref[:, pl.ds(off, w)] = sum2
                else:
                    acc_ref[:, pl.ds(off, w)] = sum2

        for gi, ((d0, d1, d2), off, w) in enumerate(GROUPS):
            wait_recv(gi, 4)
            sum4 = (keep_ref[:, pl.ds(off, w)]
                    + p1_recv[gi][0, :, :].astype(jnp.float32) * 8192.0)
            p2_send[gi][0, :, :] = sum4.astype(jnp.bfloat16)
            start(gi, 6, d2)
        for gi, ((d0, d1, d2), off, w) in enumerate(GROUPS):
            wait_recv(gi, 5)
            acc_ref[:, pl.ds(off, w)] = (
                acc_ref[:, pl.ds(off, w)]
                + p1_recv[gi][1, :, :].astype(jnp.float32) * 8192.0)

        for gi, ((d0, d1, d2), off, w) in enumerate(GROUPS):
            wait_recv(gi, 6)
            final = (acc_ref[:, pl.ds(off, w)]
                     + p2_recv[gi][0, :, :].astype(jnp.float32))
            out_ref[:, pl.ds(off, w)] = silu(final)

        for rdma in started:
            rdma.wait_send()

        @functools.partial(
            pl.run_scoped, second_barrier=pltpu.SemaphoreType.REGULAR)
        def _(second_barrier):
            for e in (1, 2, 4):
                pl.semaphore_signal(
                    second_barrier, inc=1,
                    device_id=(partner(e),),
                    device_id_type=pl.DeviceIdType.MESH,
                )
            pl.semaphore_wait(second_barrier, 3)

    p0_shapes = [
        pltpu.VMEM((4, m_per, w), jnp.int8) for _, _, w in GROUPS
    ]
    p1_shapes = [
        pltpu.VMEM((2, m_per, w), jnp.int8) for _, _, w in GROUPS
    ]
    p2_shapes = [
        pltpu.VMEM((1, m_per, w), jnp.bfloat16) for _, _, w in GROUPS
    ]
    return pl.pallas_call(
        body,
        out_shape=jax.ShapeDtypeStruct((m_per, n), jnp.float32),
        in_specs=[
            pl.BlockSpec(memory_space=pltpu.VMEM),
            pl.BlockSpec(memory_space=pltpu.VMEM),
            pl.BlockSpec(memory_space=pltpu.SMEM),
            pl.BlockSpec(memory_space=pltpu.SMEM),
        ],
        out_specs=pl.BlockSpec(memory_space=pltpu.VMEM),
        scratch_shapes=(p0_shapes + p0_shapes + p1_shapes + p1_shapes
                        + p2_shapes + p2_shapes) + [
            pltpu.VMEM((m_per, n), jnp.float32),
            pltpu.VMEM((m_per, n), jnp.float32),
            pltpu.SemaphoreType.DMA((3, 7)),
            pltpu.SemaphoreType.DMA((3, 7)),
        ],
        compiler_params=pltpu.CompilerParams(
            collective_id=0,
            vmem_limit_bytes=100 * 1024 * 1024,
        ),
    )(x, w_mat, scale_x, scale_w)
